# baseline (speedup 1.0000x reference)
import jax
import jax.numpy as jnp
from jax.experimental import pallas as pl

N = 10000
A = 18
H = 128


def _gcn_conv(x, edge_index, W, b, n_nodes):
    loop = jnp.arange(n_nodes, dtype=edge_index.dtype)
    src = jnp.concatenate([edge_index[0], loop])
    dst = jnp.concatenate([edge_index[1], loop])
    ones = jnp.ones(src.shape[0], dtype=x.dtype)
    deg = jnp.zeros((n_nodes,), dtype=x.dtype).at[dst].add(ones)
    dinv = jax.lax.rsqrt(deg)
    norm = dinv[src] * dinv[dst]
    h = x @ W
    msg = h[src] * norm[:, None]
    out = jax.ops.segment_sum(msg, dst, num_segments=n_nodes)
    return out + b


def _feat(x, edge_index, W1, b1, W2, b2):
    h = _gcn_conv(x, edge_index, W1, b1, x.shape[0])
    h = jax.nn.elu(h)
    h = _gcn_conv(h, edge_index, W2, b2, x.shape[0])
    return jnp.mean(h, axis=0, keepdims=True)


def _head_kernel(fs_ref, fns_ref, oh_ref, Wf_ref, bf_ref, Wi1_ref, bi1_ref,
                 Wi2_ref, bi2_ref, Wi3_ref, bi3_ref, ns_ref, ah_ref):
    fs = fs_ref[...]
    fns = fns_ref[...]
    fm_in = jnp.concatenate([oh_ref[...], fs], axis=-1)
    ns_ref[...] = fm_in @ Wf_ref[...] + bf_ref[...][None, :]
    im_in = jnp.concatenate([fs, fns], axis=-1)
    h = jnp.maximum(im_in @ Wi1_ref[...] + bi1_ref[...][None, :], 0.0)
    h = jnp.maximum(h @ Wi2_ref[...] + bi2_ref[...][None, :], 0.0)
    ah_ref[...] = h @ Wi3_ref[...] + bi3_ref[...][None, :]


def kernel(x_s, edge_index_s, x_ns, edge_index_ns, action, W1, b1, W2, b2,
           Wf, bf, Wi1, bi1, Wi2, bi2, Wi3, bi3):
    f_state = _feat(x_s, edge_index_s, W1, b1, W2, b2)
    f_next_state = _feat(x_ns, edge_index_ns, W1, b1, W2, b2)
    onehot = jax.nn.one_hot(action, A, dtype=f_state.dtype)
    next_state_hat, action_hat = pl.pallas_call(
        _head_kernel,
        out_shape=(
            jax.ShapeDtypeStruct((1, H), jnp.float32),
            jax.ShapeDtypeStruct((1, A), jnp.float32),
        ),
    )(f_state, f_next_state, onehot, Wf, bf, Wi1, bi1, Wi2, bi2, Wi3, bi3)
    return (f_next_state, next_state_hat, action_hat)


# speedup vs baseline: 19.7931x; 19.7931x over previous
import jax
import jax.numpy as jnp
from jax import lax
from jax.experimental import pallas as pl
from jax.experimental.pallas import tpu as pltpu
from jax.experimental.pallas import tpu_sc as plsc

N = 10000
E = 320000
D = 128
H = 128
A = 18

NC = 2
NS = 16
B = 80
RPT = E // (NS * B)
NP = 10240
NPT = NP // NS
ZR = 128
FW = 64

_mesh = plsc.VectorSubcoreMesh(core_axis_name="c", subcore_axis_name="s")
_sc_params = pltpu.CompilerParams(use_tc_tiling_on_sc=False)


def _l1_body(dst_hbm, ones_hbm, zerosf_hbm, deg_out,
             dst_v, ones_v, stage_v, acc_sh, sem):
    c = lax.axis_index("c")
    s = lax.axis_index("s")
    pltpu.sync_copy(zerosf_hbm, stage_v)
    pltpu.sync_copy(stage_v, acc_sh.at[pl.ds(s * NPT, NPT)])
    pltpu.sync_copy(ones_hbm, ones_v)
    pltpu.sync_copy(dst_hbm.at[c, s], dst_v)
    plsc.subcore_barrier()

    @pl.loop(0, RPT)
    def _(j):
        pltpu.sync_copy(ones_v, acc_sh.at[dst_v.at[j]], add=True)

    plsc.subcore_barrier()
    pltpu.sync_copy(acc_sh.at[pl.ds(s * NPT, NPT)], stage_v)
    pltpu.sync_copy(stage_v, deg_out.at[c, s])


def _l2_body(src_hbm, dst_hbm, gsl, gsh, gnl, gnh, zeros_hbm,
             lo_out, hi_out,
             src_v, dst_v, rows_v, z_v, acc_sh, sem):
    c = lax.axis_index("c")
    s = lax.axis_index("s")
    for gi in range(2):
        pltpu.sync_copy(zeros_hbm, z_v)
        for z in range(NPT // ZR):
            pltpu.sync_copy(z_v, acc_sh.at[pl.ds(s * NPT + z * ZR, ZR)])
        pltpu.sync_copy(src_hbm.at[gi, s], src_v)
        pltpu.sync_copy(dst_hbm.at[gi, s], dst_v)
        plsc.subcore_barrier()

        def run(tab):
            @pl.loop(0, RPT)
            def _(j):
                pltpu.async_copy(tab.at[src_v.at[j]], rows_v, sem).wait()
                pltpu.sync_copy(rows_v, acc_sh.at[dst_v.at[j]], add=True)

        @pl.when(c == 0)
        def _():
            run(gsl if gi == 0 else gnl)

        @pl.when(c == 1)
        def _():
            run(gsh if gi == 0 else gnh)

        plsc.subcore_barrier()
        for z in range(NPT // ZR):
            pltpu.sync_copy(acc_sh.at[pl.ds(s * NPT + z * ZR, ZR)], z_v)

            @pl.when(c == 0)
            def _():
                pltpu.sync_copy(z_v, lo_out.at[gi, s, pl.ds(z * ZR, ZR)])

            @pl.when(c == 1)
            def _():
                pltpu.sync_copy(z_v, hi_out.at[gi, s, pl.ds(z * ZR, ZR)])

        plsc.subcore_barrier()


def _l3_body(src_hbm, dst_hbm, dv_hbm, zerosf_hbm,
             cc_out,
             src_v, dst_v, vals_v, stage_v, dv_sh, cc_sh, sem):
    c = lax.axis_index("c")
    s = lax.axis_index("s")
    pltpu.sync_copy(dv_hbm.at[c, pl.ds(s * NPT, NPT)], stage_v)
    pltpu.sync_copy(stage_v, dv_sh.at[pl.ds(s * NPT, NPT)])
    pltpu.sync_copy(zerosf_hbm, stage_v)
    pltpu.sync_copy(stage_v, cc_sh.at[pl.ds(s * NPT, NPT)])
    pltpu.sync_copy(src_hbm.at[c, s], src_v)
    pltpu.sync_copy(dst_hbm.at[c, s], dst_v)
    plsc.subcore_barrier()

    @pl.loop(0, RPT)
    def _(j):
        pltpu.async_copy(dv_sh.at[dst_v.at[j]], vals_v, sem).wait()
        pltpu.sync_copy(vals_v, cc_sh.at[src_v.at[j]], add=True)

    plsc.subcore_barrier()
    pltpu.sync_copy(cc_sh.at[pl.ds(s * NPT, NPT)], stage_v)
    pltpu.sync_copy(stage_v, cc_out.at[c, s])


def _l1(dst4d, onesf, zerosf):
    k = pl.kernel(
        _l1_body,
        out_type=jax.ShapeDtypeStruct((NC, NS, NPT), jnp.float32),
        mesh=_mesh,
        compiler_params=_sc_params,
        scratch_types=[
            pltpu.VMEM((RPT, B), jnp.int32),
            pltpu.VMEM((B,), jnp.float32),
            pltpu.VMEM((NPT,), jnp.float32),
            pltpu.VMEM_SHARED((NP,), jnp.float32),
            pltpu.SemaphoreType.DMA,
        ],
    )
    return k(dst4d, onesf, zerosf)


def _l2(src4d, dst4d, gsl, gsh, gnl, gnh, zeros):
    k = pl.kernel(
        _l2_body,
        out_type=(
            jax.ShapeDtypeStruct((NC, NS, NPT, FW), jnp.float32),
            jax.ShapeDtypeStruct((NC, NS, NPT, FW), jnp.float32),
        ),
        mesh=_mesh,
        compiler_params=_sc_params,
        scratch_types=[
            pltpu.VMEM((RPT, B), jnp.int32),
            pltpu.VMEM((RPT, B), jnp.int32),
            pltpu.VMEM((B, FW), jnp.float32),
            pltpu.VMEM((ZR, FW), jnp.float32),
            pltpu.VMEM_SHARED((NP, FW), jnp.float32),
            pltpu.SemaphoreType.DMA,
        ],
    )
    return k(src4d, dst4d, gsl, gsh, gnl, gnh, zeros)


def _l3(src4d, dst4d, dv2, zerosf):
    k = pl.kernel(
        _l3_body,
        out_type=jax.ShapeDtypeStruct((NC, NS, NPT), jnp.float32),
        mesh=_mesh,
        compiler_params=_sc_params,
        scratch_types=[
            pltpu.VMEM((RPT, B), jnp.int32),
            pltpu.VMEM((RPT, B), jnp.int32),
            pltpu.VMEM((B,), jnp.float32),
            pltpu.VMEM((NPT,), jnp.float32),
            pltpu.VMEM_SHARED((NP,), jnp.float32),
            pltpu.VMEM_SHARED((NP,), jnp.float32),
            pltpu.SemaphoreType.DMA,
        ],
    )
    return k(src4d, dst4d, dv2, zerosf)


def _t1_body(x_ref, W1_ref, deg_ref, glo_ref, ghi_ref, dv1_ref):
    dinv1 = lax.rsqrt(deg_ref[...] + 1.0)
    h = jnp.dot(x_ref[...], W1_ref[...],
                preferred_element_type=jnp.float32,
                precision=lax.Precision.HIGHEST)
    g = dinv1 * h
    glo_ref[...] = g[:, :FW]
    ghi_ref[...] = g[:, FW:]
    dv1_ref[...] = dinv1


def _t1(x, W1, deg):
    return pl.pallas_call(
        _t1_body,
        out_shape=(
            jax.ShapeDtypeStruct((N, FW), jnp.float32),
            jax.ShapeDtypeStruct((N, FW), jnp.float32),
            jax.ShapeDtypeStruct((N, 1), jnp.float32),
        ),
    )(x, W1, deg)


def _t2_body(tlo_ref, thi_ref, glo_ref, ghi_ref, dv_ref, dvr_ref, ccr_ref,
             b1_ref, s_ref):
    dinv = dv_ref[...]
    tmp = jnp.concatenate([tlo_ref[...], thi_ref[...]], axis=-1)
    g = jnp.concatenate([glo_ref[...], ghi_ref[...]], axis=-1)
    v = dinv * (tmp + g) + b1_ref[...]
    h1 = jnp.where(v > 0.0, v, jnp.exp(jnp.minimum(v, 0.0)) - 1.0)
    dvr = dvr_ref[...]
    cvec_row = dvr * (ccr_ref[...] + dvr)
    s_ref[...] = jnp.dot(cvec_row, h1,
                         preferred_element_type=jnp.float32,
                         precision=lax.Precision.HIGHEST)


def _t2(tlo, thi, glo, ghi, dv1, dvr, ccr, b1):
    return pl.pallas_call(
        _t2_body,
        out_shape=jax.ShapeDtypeStruct((1, H), jnp.float32),
    )(tlo, thi, glo, ghi, dv1, dvr, ccr, b1)


def _t3_body(ss_ref, sns_ref, W2_ref, b2_ref, oh_ref, Wf_ref, bf_ref,
             Wi1_ref, bi1_ref, Wi2_ref, bi2_ref, Wi3_ref, bi3_ref,
             fns_ref, nsh_ref, ah_ref):
    hi = lax.Precision.HIGHEST
    f_s = jnp.dot(ss_ref[...] * (1.0 / N), W2_ref[...], precision=hi) + b2_ref[...]
    f_ns = jnp.dot(sns_ref[...] * (1.0 / N), W2_ref[...], precision=hi) + b2_ref[...]
    fns_ref[...] = f_ns
    fm_in = jnp.concatenate([oh_ref[...], f_s], axis=-1)
    nsh_ref[...] = jnp.dot(fm_in, Wf_ref[...], precision=hi) + bf_ref[...]
    im_in = jnp.concatenate([f_s, f_ns], axis=-1)
    hh = jnp.maximum(jnp.dot(im_in, Wi1_ref[...], precision=hi) + bi1_ref[...], 0.0)
    hh = jnp.maximum(jnp.dot(hh, Wi2_ref[...], precision=hi) + bi2_ref[...], 0.0)
    ah_ref[...] = jnp.dot(hh, Wi3_ref[...], precision=hi) + bi3_ref[...]


def _t3(s_s, s_ns, W2, b2, onehot, Wf, bf, Wi1, bi1, Wi2, bi2, Wi3, bi3):
    return pl.pallas_call(
        _t3_body,
        out_shape=(
            jax.ShapeDtypeStruct((1, H), jnp.float32),
            jax.ShapeDtypeStruct((1, H), jnp.float32),
            jax.ShapeDtypeStruct((1, A), jnp.float32),
        ),
    )(s_s, s_ns, W2, b2.reshape(1, H), onehot, Wf, bf.reshape(1, H),
      Wi1, bi1.reshape(1, -1), Wi2, bi2.reshape(1, -1), Wi3, bi3.reshape(1, -1))


def kernel(x_s, edge_index_s, x_ns, edge_index_ns, action, W1, b1, W2, b2,
           Wf, bf, Wi1, bi1, Wi2, bi2, Wi3, bi3):
    i32 = jnp.int32
    src4d = jnp.stack([edge_index_s[0].astype(i32).reshape(NS, RPT, B),
                       edge_index_ns[0].astype(i32).reshape(NS, RPT, B)])
    dst4d = jnp.stack([edge_index_s[1].astype(i32).reshape(NS, RPT, B),
                       edge_index_ns[1].astype(i32).reshape(NS, RPT, B)])
    onesf = jnp.ones((B,), jnp.float32)
    zeros = jnp.zeros((ZR, FW), jnp.float32)
    zerosf = jnp.zeros((NPT,), jnp.float32)

    deg = _l1(dst4d, onesf, zerosf).reshape(NC, NP)
    gsl, gsh, dv_s = _t1(x_s, W1, deg[0, :N, None])
    gnl, gnh, dv_ns = _t1(x_ns, W1, deg[1, :N, None])
    dv2 = jnp.stack([jnp.pad(dv_s[:, 0], (0, NP - N)),
                     jnp.pad(dv_ns[:, 0], (0, NP - N))])
    tmp_lo, tmp_hi = _l2(src4d, dst4d, gsl, gsh, gnl, gnh, zeros)
    tmp_lo = tmp_lo.reshape(NC, NP, FW)
    tmp_hi = tmp_hi.reshape(NC, NP, FW)
    cc = _l3(src4d, dst4d, dv2, zerosf).reshape(NC, NP)
    b1r = b1.reshape(1, H)
    s_s = _t2(tmp_lo[0, :N], tmp_hi[0, :N], gsl, gsh, dv_s,
              dv_s.reshape(1, N), cc[0, :N].reshape(1, N), b1r)
    s_ns = _t2(tmp_lo[1, :N], tmp_hi[1, :N], gnl, gnh, dv_ns,
               dv_ns.reshape(1, N), cc[1, :N].reshape(1, N), b1r)

    onehot = jax.nn.one_hot(action, A, dtype=jnp.float32)
    f_ns, nsh, ah = _t3(s_s, s_ns, W2, b2, onehot,
                        Wf, bf, Wi1, bi1, Wi2, bi2, Wi3, bi3)
    return (f_ns, nsh, ah)


# speedup vs baseline: 29.4637x; 1.4886x over previous
import jax
import jax.numpy as jnp
from jax import lax
from jax.experimental import pallas as pl
from jax.experimental.pallas import tpu as pltpu
from jax.experimental.pallas import tpu_sc as plsc

N = 10000
E = 320000
D = 128
H = 128
A = 18

NC = 2
NS = 16
B = 80
RPT = E // (NS * B)
NP = 10240
NPT = NP // NS
ZR = 128
FW = 64
EPT = E // NS

_mesh = plsc.VectorSubcoreMesh(core_axis_name="c", subcore_axis_name="s")
_sc_params = pltpu.CompilerParams(use_tc_tiling_on_sc=False)


def _l1_body(dst_hbm, ones_hbm, zerosf_hbm, deg_out,
             dst_v, ones_v, stage_v, acc_sh, sem):
    c = lax.axis_index("c")
    s = lax.axis_index("s")
    pltpu.sync_copy(zerosf_hbm, stage_v)
    pltpu.sync_copy(stage_v, acc_sh.at[pl.ds(s * NPT, NPT)])
    pltpu.sync_copy(ones_hbm, ones_v)
    pltpu.sync_copy(dst_hbm.at[c, s], dst_v)
    plsc.subcore_barrier()
    pltpu.sync_copy(ones_v, acc_sh.at[dst_v], add=True)
    plsc.subcore_barrier()
    pltpu.sync_copy(acc_sh.at[pl.ds(s * NPT, NPT)], stage_v)
    pltpu.sync_copy(stage_v, deg_out.at[c, s])


def _l2_body(src_hbm, dst_hbm, gsl, gsh, gnl, gnh, zeros_hbm,
             lo_out, hi_out,
             src_v, dst_v, rows0_v, rows1_v, z_v, acc_sh, sem0, sem1):
    c = lax.axis_index("c")
    s = lax.axis_index("s")
    for gi in range(2):
        pltpu.sync_copy(zeros_hbm, z_v)
        for z in range(NPT // ZR):
            pltpu.sync_copy(z_v, acc_sh.at[pl.ds(s * NPT + z * ZR, ZR)])
        pltpu.sync_copy(src_hbm.at[gi, s], src_v)
        pltpu.sync_copy(dst_hbm.at[gi, s], dst_v)
        plsc.subcore_barrier()

        def run(tab):
            dummy = tab.at[pl.ds(0, B)]
            pltpu.async_copy(tab.at[src_v.at[0]], rows0_v, sem0)
            pltpu.async_copy(tab.at[src_v.at[1]], rows1_v, sem1)

            @pl.loop(0, RPT, step=2)
            def _(j):
                pltpu.make_async_copy(dummy, rows0_v, sem0).wait()
                pltpu.sync_copy(rows0_v, acc_sh.at[dst_v.at[j]], add=True)

                @pl.when(j + 2 < RPT)
                def _():
                    pltpu.async_copy(tab.at[src_v.at[j + 2]], rows0_v, sem0)

                pltpu.make_async_copy(dummy, rows1_v, sem1).wait()
                pltpu.sync_copy(rows1_v, acc_sh.at[dst_v.at[j + 1]], add=True)

                @pl.when(j + 3 < RPT)
                def _():
                    pltpu.async_copy(tab.at[src_v.at[j + 3]], rows1_v, sem1)

        @pl.when(c == 0)
        def _():
            run(gsl if gi == 0 else gnl)

        @pl.when(c == 1)
        def _():
            run(gsh if gi == 0 else gnh)

        plsc.subcore_barrier()
        for z in range(NPT // ZR):
            pltpu.sync_copy(acc_sh.at[pl.ds(s * NPT + z * ZR, ZR)], z_v)

            @pl.when(c == 0)
            def _():
                pltpu.sync_copy(z_v, lo_out.at[gi, s, pl.ds(z * ZR, ZR)])

            @pl.when(c == 1)
            def _():
                pltpu.sync_copy(z_v, hi_out.at[gi, s, pl.ds(z * ZR, ZR)])

        plsc.subcore_barrier()


def _l3_body(src_hbm, dst_hbm, dv_hbm, zerosf_hbm,
             cc_out,
             src_v, dst_v, vals_v, stage_v, dv_sh, cc_sh, sem):
    c = lax.axis_index("c")
    s = lax.axis_index("s")
    pltpu.sync_copy(dv_hbm.at[c, pl.ds(s * NPT, NPT)], stage_v)
    pltpu.sync_copy(stage_v, dv_sh.at[pl.ds(s * NPT, NPT)])
    pltpu.sync_copy(zerosf_hbm, stage_v)
    pltpu.sync_copy(stage_v, cc_sh.at[pl.ds(s * NPT, NPT)])
    pltpu.sync_copy(src_hbm.at[c, s], src_v)
    pltpu.sync_copy(dst_hbm.at[c, s], dst_v)
    plsc.subcore_barrier()
    pltpu.async_copy(dv_sh.at[dst_v], vals_v, sem).wait()
    pltpu.sync_copy(vals_v, cc_sh.at[src_v], add=True)
    plsc.subcore_barrier()
    pltpu.sync_copy(cc_sh.at[pl.ds(s * NPT, NPT)], stage_v)
    pltpu.sync_copy(stage_v, cc_out.at[c, s])


def _l1(dst3, onesf, zerosf):
    k = pl.kernel(
        _l1_body,
        out_type=jax.ShapeDtypeStruct((NC, NS, NPT), jnp.float32),
        mesh=_mesh,
        compiler_params=_sc_params,
        scratch_types=[
            pltpu.VMEM((EPT,), jnp.int32),
            pltpu.VMEM((EPT,), jnp.float32),
            pltpu.VMEM((NPT,), jnp.float32),
            pltpu.VMEM_SHARED((NP,), jnp.float32),
            pltpu.SemaphoreType.DMA,
        ],
    )
    return k(dst3, onesf, zerosf)


def _l2(src4d, dst4d, gsl, gsh, gnl, gnh, zeros):
    k = pl.kernel(
        _l2_body,
        out_type=(
            jax.ShapeDtypeStruct((NC, NS, NPT, FW), jnp.float32),
            jax.ShapeDtypeStruct((NC, NS, NPT, FW), jnp.float32),
        ),
        mesh=_mesh,
        compiler_params=_sc_params,
        scratch_types=[
            pltpu.VMEM((RPT, B), jnp.int32),
            pltpu.VMEM((RPT, B), jnp.int32),
            pltpu.VMEM((B, FW), jnp.float32),
            pltpu.VMEM((B, FW), jnp.float32),
            pltpu.VMEM((ZR, FW), jnp.float32),
            pltpu.VMEM_SHARED((NP, FW), jnp.float32),
            pltpu.SemaphoreType.DMA,
            pltpu.SemaphoreType.DMA,
        ],
    )
    return k(src4d, dst4d, gsl, gsh, gnl, gnh, zeros)


def _l3(src3, dst3, dv2, zerosf):
    k = pl.kernel(
        _l3_body,
        out_type=jax.ShapeDtypeStruct((NC, NS, NPT), jnp.float32),
        mesh=_mesh,
        compiler_params=_sc_params,
        scratch_types=[
            pltpu.VMEM((EPT,), jnp.int32),
            pltpu.VMEM((EPT,), jnp.int32),
            pltpu.VMEM((EPT,), jnp.float32),
            pltpu.VMEM((NPT,), jnp.float32),
            pltpu.VMEM_SHARED((NP,), jnp.float32),
            pltpu.VMEM_SHARED((NP,), jnp.float32),
            pltpu.SemaphoreType.DMA,
        ],
    )
    return k(src3, dst3, dv2, zerosf)


def _t0_body(x_ref, W1_ref, h_ref):
    h_ref[...] = jnp.dot(x_ref[...], W1_ref[...],
                         preferred_element_type=jnp.float32,
                         precision=lax.Precision.HIGHEST)


def _t0(x, W1):
    return pl.pallas_call(
        _t0_body,
        out_shape=jax.ShapeDtypeStruct((N, D), jnp.float32),
    )(x, W1)


def _t1_body(h_ref, deg_ref, glo_ref, ghi_ref, dv1_ref):
    dinv1 = lax.rsqrt(deg_ref[...] + 1.0)
    g = dinv1 * h_ref[...]
    glo_ref[...] = g[:, :FW]
    ghi_ref[...] = g[:, FW:]
    dv1_ref[...] = dinv1


def _t1(h, deg):
    return pl.pallas_call(
        _t1_body,
        out_shape=(
            jax.ShapeDtypeStruct((N, FW), jnp.float32),
            jax.ShapeDtypeStruct((N, FW), jnp.float32),
            jax.ShapeDtypeStruct((N, 1), jnp.float32),
        ),
    )(h, deg)


def _t2_body(tlo_ref, thi_ref, glo_ref, ghi_ref, dv_ref, dvr_ref, ccr_ref,
             b1_ref, s_ref):
    dinv = dv_ref[...]
    tmp = jnp.concatenate([tlo_ref[...], thi_ref[...]], axis=-1)
    g = jnp.concatenate([glo_ref[...], ghi_ref[...]], axis=-1)
    v = dinv * (tmp + g) + b1_ref[...]
    h1 = jnp.where(v > 0.0, v, jnp.exp(jnp.minimum(v, 0.0)) - 1.0)
    dvr = dvr_ref[...]
    cvec_row = dvr * (ccr_ref[...] + dvr)
    s_ref[...] = jnp.dot(cvec_row, h1,
                         preferred_element_type=jnp.float32,
                         precision=lax.Precision.HIGHEST)


def _t2(tlo, thi, glo, ghi, dv1, dvr, ccr, b1):
    return pl.pallas_call(
        _t2_body,
        out_shape=jax.ShapeDtypeStruct((1, H), jnp.float32),
    )(tlo, thi, glo, ghi, dv1, dvr, ccr, b1)


def _t3_body(ss_ref, sns_ref, W2_ref, b2_ref, oh_ref, Wf_ref, bf_ref,
             Wi1_ref, bi1_ref, Wi2_ref, bi2_ref, Wi3_ref, bi3_ref,
             fns_ref, nsh_ref, ah_ref):
    hi = lax.Precision.HIGHEST
    f_s = jnp.dot(ss_ref[...] * (1.0 / N), W2_ref[...], precision=hi) + b2_ref[...]
    f_ns = jnp.dot(sns_ref[...] * (1.0 / N), W2_ref[...], precision=hi) + b2_ref[...]
    fns_ref[...] = f_ns
    fm_in = jnp.concatenate([oh_ref[...], f_s], axis=-1)
    nsh_ref[...] = jnp.dot(fm_in, Wf_ref[...], precision=hi) + bf_ref[...]
    im_in = jnp.concatenate([f_s, f_ns], axis=-1)
    hh = jnp.maximum(jnp.dot(im_in, Wi1_ref[...], precision=hi) + bi1_ref[...], 0.0)
    hh = jnp.maximum(jnp.dot(hh, Wi2_ref[...], precision=hi) + bi2_ref[...], 0.0)
    ah_ref[...] = jnp.dot(hh, Wi3_ref[...], precision=hi) + bi3_ref[...]


def _t3(s_s, s_ns, W2, b2, onehot, Wf, bf, Wi1, bi1, Wi2, bi2, Wi3, bi3):
    return pl.pallas_call(
        _t3_body,
        out_shape=(
            jax.ShapeDtypeStruct((1, H), jnp.float32),
            jax.ShapeDtypeStruct((1, H), jnp.float32),
            jax.ShapeDtypeStruct((1, A), jnp.float32),
        ),
    )(s_s, s_ns, W2, b2.reshape(1, H), onehot, Wf, bf.reshape(1, H),
      Wi1, bi1.reshape(1, -1), Wi2, bi2.reshape(1, -1), Wi3, bi3.reshape(1, -1))


def kernel(x_s, edge_index_s, x_ns, edge_index_ns, action, W1, b1, W2, b2,
           Wf, bf, Wi1, bi1, Wi2, bi2, Wi3, bi3):
    i32 = jnp.int32
    src4d = jnp.stack([edge_index_s[0].astype(i32).reshape(NS, RPT, B),
                       edge_index_ns[0].astype(i32).reshape(NS, RPT, B)])
    dst4d = jnp.stack([edge_index_s[1].astype(i32).reshape(NS, RPT, B),
                       edge_index_ns[1].astype(i32).reshape(NS, RPT, B)])
    src3 = src4d.reshape(NC, NS, EPT)
    dst3 = dst4d.reshape(NC, NS, EPT)
    onesf = jnp.ones((EPT,), jnp.float32)
    zeros = jnp.zeros((ZR, FW), jnp.float32)
    zerosf = jnp.zeros((NPT,), jnp.float32)

    h_s = _t0(x_s, W1)
    h_ns = _t0(x_ns, W1)
    deg = _l1(dst3, onesf, zerosf).reshape(NC, NP)
    gsl, gsh, dv_s = _t1(h_s, deg[0, :N, None])
    gnl, gnh, dv_ns = _t1(h_ns, deg[1, :N, None])
    dv2 = jnp.stack([jnp.pad(dv_s[:, 0], (0, NP - N)),
                     jnp.pad(dv_ns[:, 0], (0, NP - N))])
    tmp_lo, tmp_hi = _l2(src4d, dst4d, gsl, gsh, gnl, gnh, zeros)
    tmp_lo = tmp_lo.reshape(NC, NP, FW)
    tmp_hi = tmp_hi.reshape(NC, NP, FW)
    cc = _l3(src3, dst3, dv2, zerosf).reshape(NC, NP)
    b1r = b1.reshape(1, H)
    s_s = _t2(tmp_lo[0, :N], tmp_hi[0, :N], gsl, gsh, dv_s,
              dv_s.reshape(1, N), cc[0, :N].reshape(1, N), b1r)
    s_ns = _t2(tmp_lo[1, :N], tmp_hi[1, :N], gnl, gnh, dv_ns,
               dv_ns.reshape(1, N), cc[1, :N].reshape(1, N), b1r)

    onehot = jax.nn.one_hot(action, A, dtype=jnp.float32)
    f_ns, nsh, ah = _t3(s_s, s_ns, W2, b2, onehot,
                        Wf, bf, Wi1, bi1, Wi2, bi2, Wi3, bi3)
    return (f_ns, nsh, ah)


# speedup vs baseline: 33.6817x; 1.1432x over previous
import jax
import jax.numpy as jnp
from jax import lax
from jax.experimental import pallas as pl
from jax.experimental.pallas import tpu as pltpu
from jax.experimental.pallas import tpu_sc as plsc

N = 10000
E = 320000
D = 128
H = 128
A = 18

NC = 2
NS = 16
B = 125
RPT = E // (NS * B)
NP = 10240
NPT = NP // NS
ZR = 128
FW = 64
EPT = E // NS

_mesh = plsc.VectorSubcoreMesh(core_axis_name="c", subcore_axis_name="s")
_sc_params = pltpu.CompilerParams(use_tc_tiling_on_sc=False)


def _l1_body(dst_hbm, ones_hbm, zerosf_hbm, deg_out,
             dst_v, ones_v, stage_v, acc_sh, sem):
    c = lax.axis_index("c")
    s = lax.axis_index("s")
    pltpu.sync_copy(zerosf_hbm, stage_v)
    pltpu.sync_copy(stage_v, acc_sh.at[pl.ds(s * NPT, NPT)])
    pltpu.sync_copy(ones_hbm, ones_v)
    pltpu.sync_copy(dst_hbm.at[c, s], dst_v)
    plsc.subcore_barrier()
    pltpu.sync_copy(ones_v, acc_sh.at[dst_v], add=True)
    plsc.subcore_barrier()
    pltpu.sync_copy(acc_sh.at[pl.ds(s * NPT, NPT)], stage_v)
    pltpu.sync_copy(stage_v, deg_out.at[c, s])


def _l2_body(src_hbm, dst_hbm, gsl, gsh, gnl, gnh, zeros_hbm,
             lo_out, hi_out,
             src_v, dst_v, rows0_v, rows1_v, z_v, acc_sh, sem0, sem1):
    c = lax.axis_index("c")
    s = lax.axis_index("s")
    for gi in range(2):
        pltpu.sync_copy(zeros_hbm, z_v)
        for z in range(NPT // ZR):
            pltpu.sync_copy(z_v, acc_sh.at[pl.ds(s * NPT + z * ZR, ZR)])
        pltpu.sync_copy(src_hbm.at[gi, s], src_v)
        pltpu.sync_copy(dst_hbm.at[gi, s], dst_v)
        plsc.subcore_barrier()

        def run(tab):
            dummy = tab.at[pl.ds(0, B)]
            pltpu.async_copy(tab.at[src_v.at[0]], rows0_v, sem0)
            pltpu.async_copy(tab.at[src_v.at[1]], rows1_v, sem1)

            @pl.loop(0, RPT, step=2)
            def _(j):
                pltpu.make_async_copy(dummy, rows0_v, sem0).wait()
                pltpu.sync_copy(rows0_v, acc_sh.at[dst_v.at[j]], add=True)

                @pl.when(j + 2 < RPT)
                def _():
                    pltpu.async_copy(tab.at[src_v.at[j + 2]], rows0_v, sem0)

                pltpu.make_async_copy(dummy, rows1_v, sem1).wait()
                pltpu.sync_copy(rows1_v, acc_sh.at[dst_v.at[j + 1]], add=True)

                @pl.when(j + 3 < RPT)
                def _():
                    pltpu.async_copy(tab.at[src_v.at[j + 3]], rows1_v, sem1)

        @pl.when(c == 0)
        def _():
            run(gsl if gi == 0 else gnl)

        @pl.when(c == 1)
        def _():
            run(gsh if gi == 0 else gnh)

        plsc.subcore_barrier()
        for z in range(NPT // ZR):
            pltpu.sync_copy(acc_sh.at[pl.ds(s * NPT + z * ZR, ZR)], z_v)

            @pl.when(c == 0)
            def _():
                pltpu.sync_copy(z_v, lo_out.at[gi, s, pl.ds(z * ZR, ZR)])

            @pl.when(c == 1)
            def _():
                pltpu.sync_copy(z_v, hi_out.at[gi, s, pl.ds(z * ZR, ZR)])

        plsc.subcore_barrier()


def _l3_body(src_hbm, dst_hbm, dv_hbm, zerosf_hbm,
             cc_out,
             src_v, dst_v, vals_v, stage_v, dv_sh, cc_sh, sem):
    c = lax.axis_index("c")
    s = lax.axis_index("s")
    pltpu.sync_copy(dv_hbm.at[c, pl.ds(s * NPT, NPT)], stage_v)
    pltpu.sync_copy(stage_v, dv_sh.at[pl.ds(s * NPT, NPT)])
    pltpu.sync_copy(zerosf_hbm, stage_v)
    pltpu.sync_copy(stage_v, cc_sh.at[pl.ds(s * NPT, NPT)])
    pltpu.sync_copy(src_hbm.at[c, s], src_v)
    pltpu.sync_copy(dst_hbm.at[c, s], dst_v)
    plsc.subcore_barrier()
    pltpu.async_copy(dv_sh.at[dst_v], vals_v, sem).wait()
    pltpu.sync_copy(vals_v, cc_sh.at[src_v], add=True)
    plsc.subcore_barrier()
    pltpu.sync_copy(cc_sh.at[pl.ds(s * NPT, NPT)], stage_v)
    pltpu.sync_copy(stage_v, cc_out.at[c, s])


def _l1(dst3, onesf, zerosf):
    k = pl.kernel(
        _l1_body,
        out_type=jax.ShapeDtypeStruct((NC, NS, NPT), jnp.float32),
        mesh=_mesh,
        compiler_params=_sc_params,
        scratch_types=[
            pltpu.VMEM((EPT,), jnp.int32),
            pltpu.VMEM((EPT,), jnp.float32),
            pltpu.VMEM((NPT,), jnp.float32),
            pltpu.VMEM_SHARED((NP,), jnp.float32),
            pltpu.SemaphoreType.DMA,
        ],
    )
    return k(dst3, onesf, zerosf)


def _l2(src4d, dst4d, gsl, gsh, gnl, gnh, zeros):
    k = pl.kernel(
        _l2_body,
        out_type=(
            jax.ShapeDtypeStruct((NC, NS, NPT, FW), jnp.float32),
            jax.ShapeDtypeStruct((NC, NS, NPT, FW), jnp.float32),
        ),
        mesh=_mesh,
        compiler_params=_sc_params,
        scratch_types=[
            pltpu.VMEM((RPT, B), jnp.int32),
            pltpu.VMEM((RPT, B), jnp.int32),
            pltpu.VMEM((B, FW), jnp.float32),
            pltpu.VMEM((B, FW), jnp.float32),
            pltpu.VMEM((ZR, FW), jnp.float32),
            pltpu.VMEM_SHARED((NP, FW), jnp.float32),
            pltpu.SemaphoreType.DMA,
            pltpu.SemaphoreType.DMA,
        ],
    )
    return k(src4d, dst4d, gsl, gsh, gnl, gnh, zeros)


def _l3(src3, dst3, dv2, zerosf):
    k = pl.kernel(
        _l3_body,
        out_type=jax.ShapeDtypeStruct((NC, NS, NPT), jnp.float32),
        mesh=_mesh,
        compiler_params=_sc_params,
        scratch_types=[
            pltpu.VMEM((EPT,), jnp.int32),
            pltpu.VMEM((EPT,), jnp.int32),
            pltpu.VMEM((EPT,), jnp.float32),
            pltpu.VMEM((NPT,), jnp.float32),
            pltpu.VMEM_SHARED((NP,), jnp.float32),
            pltpu.VMEM_SHARED((NP,), jnp.float32),
            pltpu.SemaphoreType.DMA,
        ],
    )
    return k(src3, dst3, dv2, zerosf)


def _t0_body(x_ref, W1_ref, h_ref):
    h_ref[...] = jnp.dot(x_ref[...], W1_ref[...],
                         preferred_element_type=jnp.float32,
                         precision=lax.Precision.HIGHEST)


def _t0(x, W1):
    return pl.pallas_call(
        _t0_body,
        out_shape=jax.ShapeDtypeStruct((N, D), jnp.float32),
    )(x, W1)


def _t1_body(h_ref, deg_ref, glo_ref, ghi_ref, dv1_ref):
    dinv1 = lax.rsqrt(deg_ref[...] + 1.0)
    g = dinv1 * h_ref[...]
    glo_ref[...] = g[:, :FW]
    ghi_ref[...] = g[:, FW:]
    dv1_ref[...] = dinv1


def _t1(h, deg):
    return pl.pallas_call(
        _t1_body,
        out_shape=(
            jax.ShapeDtypeStruct((N, FW), jnp.float32),
            jax.ShapeDtypeStruct((N, FW), jnp.float32),
            jax.ShapeDtypeStruct((N, 1), jnp.float32),
        ),
    )(h, deg)


def _t2_body(tlo_ref, thi_ref, glo_ref, ghi_ref, dv_ref, dvr_ref, ccr_ref,
             b1_ref, s_ref):
    dinv = dv_ref[...]
    tmp = jnp.concatenate([tlo_ref[...], thi_ref[...]], axis=-1)
    g = jnp.concatenate([glo_ref[...], ghi_ref[...]], axis=-1)
    v = dinv * (tmp + g) + b1_ref[...]
    h1 = jnp.where(v > 0.0, v, jnp.exp(jnp.minimum(v, 0.0)) - 1.0)
    dvr = dvr_ref[...]
    cvec_row = dvr * (ccr_ref[...] + dvr)
    s_ref[...] = jnp.dot(cvec_row, h1,
                         preferred_element_type=jnp.float32,
                         precision=lax.Precision.HIGHEST)


def _t2(tlo, thi, glo, ghi, dv1, dvr, ccr, b1):
    return pl.pallas_call(
        _t2_body,
        out_shape=jax.ShapeDtypeStruct((1, H), jnp.float32),
    )(tlo, thi, glo, ghi, dv1, dvr, ccr, b1)


def _t3_body(ss_ref, sns_ref, W2_ref, b2_ref, oh_ref, Wf_ref, bf_ref,
             Wi1_ref, bi1_ref, Wi2_ref, bi2_ref, Wi3_ref, bi3_ref,
             fns_ref, nsh_ref, ah_ref):
    hi = lax.Precision.HIGHEST
    f_s = jnp.dot(ss_ref[...] * (1.0 / N), W2_ref[...], precision=hi) + b2_ref[...]
    f_ns = jnp.dot(sns_ref[...] * (1.0 / N), W2_ref[...], precision=hi) + b2_ref[...]
    fns_ref[...] = f_ns
    fm_in = jnp.concatenate([oh_ref[...], f_s], axis=-1)
    nsh_ref[...] = jnp.dot(fm_in, Wf_ref[...], precision=hi) + bf_ref[...]
    im_in = jnp.concatenate([f_s, f_ns], axis=-1)
    hh = jnp.maximum(jnp.dot(im_in, Wi1_ref[...], precision=hi) + bi1_ref[...], 0.0)
    hh = jnp.maximum(jnp.dot(hh, Wi2_ref[...], precision=hi) + bi2_ref[...], 0.0)
    ah_ref[...] = jnp.dot(hh, Wi3_ref[...], precision=hi) + bi3_ref[...]


def _t3(s_s, s_ns, W2, b2, onehot, Wf, bf, Wi1, bi1, Wi2, bi2, Wi3, bi3):
    return pl.pallas_call(
        _t3_body,
        out_shape=(
            jax.ShapeDtypeStruct((1, H), jnp.float32),
            jax.ShapeDtypeStruct((1, H), jnp.float32),
            jax.ShapeDtypeStruct((1, A), jnp.float32),
        ),
    )(s_s, s_ns, W2, b2.reshape(1, H), onehot, Wf, bf.reshape(1, H),
      Wi1, bi1.reshape(1, -1), Wi2, bi2.reshape(1, -1), Wi3, bi3.reshape(1, -1))


def kernel(x_s, edge_index_s, x_ns, edge_index_ns, action, W1, b1, W2, b2,
           Wf, bf, Wi1, bi1, Wi2, bi2, Wi3, bi3):
    i32 = jnp.int32
    src4d = jnp.stack([edge_index_s[0].astype(i32).reshape(NS, RPT, B),
                       edge_index_ns[0].astype(i32).reshape(NS, RPT, B)])
    dst4d = jnp.stack([edge_index_s[1].astype(i32).reshape(NS, RPT, B),
                       edge_index_ns[1].astype(i32).reshape(NS, RPT, B)])
    src3 = src4d.reshape(NC, NS, EPT)
    dst3 = dst4d.reshape(NC, NS, EPT)
    onesf = jnp.ones((EPT,), jnp.float32)
    zeros = jnp.zeros((ZR, FW), jnp.float32)
    zerosf = jnp.zeros((NPT,), jnp.float32)

    h_s = _t0(x_s, W1)
    h_ns = _t0(x_ns, W1)
    deg = _l1(dst3, onesf, zerosf).reshape(NC, NP)
    gsl, gsh, dv_s = _t1(h_s, deg[0, :N, None])
    gnl, gnh, dv_ns = _t1(h_ns, deg[1, :N, None])
    dv2 = jnp.stack([jnp.pad(dv_s[:, 0], (0, NP - N)),
                     jnp.pad(dv_ns[:, 0], (0, NP - N))])
    tmp_lo, tmp_hi = _l2(src4d, dst4d, gsl, gsh, gnl, gnh, zeros)
    tmp_lo = tmp_lo.reshape(NC, NP, FW)
    tmp_hi = tmp_hi.reshape(NC, NP, FW)
    cc = _l3(src3, dst3, dv2, zerosf).reshape(NC, NP)
    b1r = b1.reshape(1, H)
    s_s = _t2(tmp_lo[0, :N], tmp_hi[0, :N], gsl, gsh, dv_s,
              dv_s.reshape(1, N), cc[0, :N].reshape(1, N), b1r)
    s_ns = _t2(tmp_lo[1, :N], tmp_hi[1, :N], gnl, gnh, dv_ns,
               dv_ns.reshape(1, N), cc[1, :N].reshape(1, N), b1r)

    onehot = jax.nn.one_hot(action, A, dtype=jnp.float32)
    f_ns, nsh, ah = _t3(s_s, s_ns, W2, b2, onehot,
                        Wf, bf, Wi1, bi1, Wi2, bi2, Wi3, bi3)
    return (f_ns, nsh, ah)


# speedup vs baseline: 38.7737x; 1.1512x over previous
import jax
import jax.numpy as jnp
from jax import lax
from jax.experimental import pallas as pl
from jax.experimental.pallas import tpu as pltpu
from jax.experimental.pallas import tpu_sc as plsc

N = 10000
E = 320000
D = 128
H = 128
A = 18

NC = 2
NS = 16
B = 125
RPT = E // (NS * B)
NP = 10240
NPT = NP // NS
ZR = 128
FW = 64
EPT = E // NS
NBUF = 4

_mesh = plsc.VectorSubcoreMesh(core_axis_name="c", subcore_axis_name="s")
_sc_params = pltpu.CompilerParams(use_tc_tiling_on_sc=False)


def _l1_body(dst_hbm, ones_hbm, zerosf_hbm, deg_out,
             dst_v, ones_v, stage_v, acc_sh, sem):
    c = lax.axis_index("c")
    s = lax.axis_index("s")
    pltpu.sync_copy(zerosf_hbm, stage_v)
    pltpu.sync_copy(stage_v, acc_sh.at[pl.ds(s * NPT, NPT)])
    pltpu.sync_copy(ones_hbm, ones_v)
    pltpu.sync_copy(dst_hbm.at[c, s], dst_v)
    plsc.subcore_barrier()
    pltpu.sync_copy(ones_v, acc_sh.at[dst_v], add=True)
    plsc.subcore_barrier()
    pltpu.sync_copy(acc_sh.at[pl.ds(s * NPT, NPT)], stage_v)
    pltpu.sync_copy(stage_v, deg_out.at[c, s])


def _l2_body(src_hbm, dst_hbm, gsl, gsh, gnl, gnh, zeros_hbm,
             lo_out, hi_out,
             src_v, dst_v, rows0_v, rows1_v, rows2_v, rows3_v, z_v, acc_sh,
             sem0, sem1, sem2, sem3):
    c = lax.axis_index("c")
    s = lax.axis_index("s")
    for gi in range(2):
        pltpu.sync_copy(zeros_hbm, z_v)
        for z in range(NPT // ZR):
            pltpu.sync_copy(z_v, acc_sh.at[pl.ds(s * NPT + z * ZR, ZR)])
        pltpu.sync_copy(src_hbm.at[gi, s], src_v)
        pltpu.sync_copy(dst_hbm.at[gi, s], dst_v)
        plsc.subcore_barrier()

        def run(tab):
            dummy = tab.at[pl.ds(0, B)]
            bufs = (rows0_v, rows1_v, rows2_v, rows3_v)
            sems = (sem0, sem1, sem2, sem3)
            for k in range(NBUF):
                pltpu.async_copy(tab.at[src_v.at[k]], bufs[k], sems[k])

            @pl.loop(0, RPT, step=NBUF)
            def _(j):
                for k in range(NBUF):
                    pltpu.make_async_copy(dummy, bufs[k], sems[k]).wait()
                    pltpu.sync_copy(bufs[k], acc_sh.at[dst_v.at[j + k]],
                                    add=True)

                    @pl.when(j + NBUF + k < RPT)
                    def _(k=k):
                        pltpu.async_copy(tab.at[src_v.at[j + NBUF + k]],
                                         bufs[k], sems[k])

        @pl.when(c == 0)
        def _():
            run(gsl if gi == 0 else gnl)

        @pl.when(c == 1)
        def _():
            run(gsh if gi == 0 else gnh)

        plsc.subcore_barrier()
        for z in range(NPT // ZR):
            pltpu.sync_copy(acc_sh.at[pl.ds(s * NPT + z * ZR, ZR)], z_v)

            @pl.when(c == 0)
            def _():
                pltpu.sync_copy(z_v, lo_out.at[gi, s, pl.ds(z * ZR, ZR)])

            @pl.when(c == 1)
            def _():
                pltpu.sync_copy(z_v, hi_out.at[gi, s, pl.ds(z * ZR, ZR)])

        plsc.subcore_barrier()


def _l3_body(src_hbm, dst_hbm, dv_hbm, zerosf_hbm,
             cc_out,
             src_v, dst_v, vals_v, stage_v, dv_sh, cc_sh, sem):
    c = lax.axis_index("c")
    s = lax.axis_index("s")
    pltpu.sync_copy(dv_hbm.at[c, pl.ds(s * NPT, NPT)], stage_v)
    pltpu.sync_copy(stage_v, dv_sh.at[pl.ds(s * NPT, NPT)])
    pltpu.sync_copy(zerosf_hbm, stage_v)
    pltpu.sync_copy(stage_v, cc_sh.at[pl.ds(s * NPT, NPT)])
    pltpu.sync_copy(src_hbm.at[c, s], src_v)
    pltpu.sync_copy(dst_hbm.at[c, s], dst_v)
    plsc.subcore_barrier()
    pltpu.async_copy(dv_sh.at[dst_v], vals_v, sem).wait()
    pltpu.sync_copy(vals_v, cc_sh.at[src_v], add=True)
    plsc.subcore_barrier()
    pltpu.sync_copy(cc_sh.at[pl.ds(s * NPT, NPT)], stage_v)
    pltpu.sync_copy(stage_v, cc_out.at[c, s])


def _l1(dst3, onesf, zerosf):
    k = pl.kernel(
        _l1_body,
        out_type=jax.ShapeDtypeStruct((NC, NS, NPT), jnp.float32),
        mesh=_mesh,
        compiler_params=_sc_params,
        scratch_types=[
            pltpu.VMEM((EPT,), jnp.int32),
            pltpu.VMEM((EPT,), jnp.float32),
            pltpu.VMEM((NPT,), jnp.float32),
            pltpu.VMEM_SHARED((NP,), jnp.float32),
            pltpu.SemaphoreType.DMA,
        ],
    )
    return k(dst3, onesf, zerosf)


def _l2(src4d, dst4d, gsl, gsh, gnl, gnh, zeros):
    k = pl.kernel(
        _l2_body,
        out_type=(
            jax.ShapeDtypeStruct((NC, NS, NPT, FW), jnp.float32),
            jax.ShapeDtypeStruct((NC, NS, NPT, FW), jnp.float32),
        ),
        mesh=_mesh,
        compiler_params=_sc_params,
        scratch_types=[
            pltpu.VMEM((RPT, B), jnp.int32),
            pltpu.VMEM((RPT, B), jnp.int32),
            pltpu.VMEM((B, FW), jnp.float32),
            pltpu.VMEM((B, FW), jnp.float32),
            pltpu.VMEM((B, FW), jnp.float32),
            pltpu.VMEM((B, FW), jnp.float32),
            pltpu.VMEM((ZR, FW), jnp.float32),
            pltpu.VMEM_SHARED((NP, FW), jnp.float32),
            pltpu.SemaphoreType.DMA,
            pltpu.SemaphoreType.DMA,
            pltpu.SemaphoreType.DMA,
            pltpu.SemaphoreType.DMA,
        ],
    )
    return k(src4d, dst4d, gsl, gsh, gnl, gnh, zeros)


def _l3(src3, dst3, dv2, zerosf):
    k = pl.kernel(
        _l3_body,
        out_type=jax.ShapeDtypeStruct((NC, NS, NPT), jnp.float32),
        mesh=_mesh,
        compiler_params=_sc_params,
        scratch_types=[
            pltpu.VMEM((EPT,), jnp.int32),
            pltpu.VMEM((EPT,), jnp.int32),
            pltpu.VMEM((EPT,), jnp.float32),
            pltpu.VMEM((NPT,), jnp.float32),
            pltpu.VMEM_SHARED((NP,), jnp.float32),
            pltpu.VMEM_SHARED((NP,), jnp.float32),
            pltpu.SemaphoreType.DMA,
        ],
    )
    return k(src3, dst3, dv2, zerosf)


def _t0_body(x_ref, W1_ref, h_ref):
    h_ref[...] = jnp.dot(x_ref[...], W1_ref[...],
                         preferred_element_type=jnp.float32,
                         precision=lax.Precision.HIGHEST)


def _t0(x, W1):
    return pl.pallas_call(
        _t0_body,
        out_shape=jax.ShapeDtypeStruct((N, D), jnp.float32),
    )(x, W1)


def _t1_body(h_ref, deg_ref, glo_ref, ghi_ref, dv1_ref):
    dinv1 = lax.rsqrt(deg_ref[...] + 1.0)
    g = dinv1 * h_ref[...]
    glo_ref[...] = g[:, :FW]
    ghi_ref[...] = g[:, FW:]
    dv1_ref[...] = dinv1


def _t1(h, deg):
    return pl.pallas_call(
        _t1_body,
        out_shape=(
            jax.ShapeDtypeStruct((N, FW), jnp.float32),
            jax.ShapeDtypeStruct((N, FW), jnp.float32),
            jax.ShapeDtypeStruct((N, 1), jnp.float32),
        ),
    )(h, deg)


def _t2_body(tlo_ref, thi_ref, glo_ref, ghi_ref, dv_ref, dvr_ref, ccr_ref,
             b1_ref, s_ref):
    dinv = dv_ref[...]
    tmp = jnp.concatenate([tlo_ref[...], thi_ref[...]], axis=-1)
    g = jnp.concatenate([glo_ref[...], ghi_ref[...]], axis=-1)
    v = dinv * (tmp + g) + b1_ref[...]
    h1 = jnp.where(v > 0.0, v, jnp.exp(jnp.minimum(v, 0.0)) - 1.0)
    dvr = dvr_ref[...]
    cvec_row = dvr * (ccr_ref[...] + dvr)
    s_ref[...] = jnp.dot(cvec_row, h1,
                         preferred_element_type=jnp.float32,
                         precision=lax.Precision.HIGHEST)


def _t2(tlo, thi, glo, ghi, dv1, dvr, ccr, b1):
    return pl.pallas_call(
        _t2_body,
        out_shape=jax.ShapeDtypeStruct((1, H), jnp.float32),
    )(tlo, thi, glo, ghi, dv1, dvr, ccr, b1)


def _t3_body(ss_ref, sns_ref, W2_ref, b2_ref, oh_ref, Wf_ref, bf_ref,
             Wi1_ref, bi1_ref, Wi2_ref, bi2_ref, Wi3_ref, bi3_ref,
             fns_ref, nsh_ref, ah_ref):
    hi = lax.Precision.HIGHEST
    f_s = jnp.dot(ss_ref[...] * (1.0 / N), W2_ref[...], precision=hi) + b2_ref[...]
    f_ns = jnp.dot(sns_ref[...] * (1.0 / N), W2_ref[...], precision=hi) + b2_ref[...]
    fns_ref[...] = f_ns
    fm_in = jnp.concatenate([oh_ref[...], f_s], axis=-1)
    nsh_ref[...] = jnp.dot(fm_in, Wf_ref[...], precision=hi) + bf_ref[...]
    im_in = jnp.concatenate([f_s, f_ns], axis=-1)
    hh = jnp.maximum(jnp.dot(im_in, Wi1_ref[...], precision=hi) + bi1_ref[...], 0.0)
    hh = jnp.maximum(jnp.dot(hh, Wi2_ref[...], precision=hi) + bi2_ref[...], 0.0)
    ah_ref[...] = jnp.dot(hh, Wi3_ref[...], precision=hi) + bi3_ref[...]


def _t3(s_s, s_ns, W2, b2, onehot, Wf, bf, Wi1, bi1, Wi2, bi2, Wi3, bi3):
    return pl.pallas_call(
        _t3_body,
        out_shape=(
            jax.ShapeDtypeStruct((1, H), jnp.float32),
            jax.ShapeDtypeStruct((1, H), jnp.float32),
            jax.ShapeDtypeStruct((1, A), jnp.float32),
        ),
    )(s_s, s_ns, W2, b2.reshape(1, H), onehot, Wf, bf.reshape(1, H),
      Wi1, bi1.reshape(1, -1), Wi2, bi2.reshape(1, -1), Wi3, bi3.reshape(1, -1))


def kernel(x_s, edge_index_s, x_ns, edge_index_ns, action, W1, b1, W2, b2,
           Wf, bf, Wi1, bi1, Wi2, bi2, Wi3, bi3):
    i32 = jnp.int32
    src4d = jnp.stack([edge_index_s[0].astype(i32).reshape(NS, RPT, B),
                       edge_index_ns[0].astype(i32).reshape(NS, RPT, B)])
    dst4d = jnp.stack([edge_index_s[1].astype(i32).reshape(NS, RPT, B),
                       edge_index_ns[1].astype(i32).reshape(NS, RPT, B)])
    src3 = src4d.reshape(NC, NS, EPT)
    dst3 = dst4d.reshape(NC, NS, EPT)
    onesf = jnp.ones((EPT,), jnp.float32)
    zeros = jnp.zeros((ZR, FW), jnp.float32)
    zerosf = jnp.zeros((NPT,), jnp.float32)

    h_s = _t0(x_s, W1)
    h_ns = _t0(x_ns, W1)
    deg = _l1(dst3, onesf, zerosf).reshape(NC, NP)
    gsl, gsh, dv_s = _t1(h_s, deg[0, :N, None])
    gnl, gnh, dv_ns = _t1(h_ns, deg[1, :N, None])
    dv2 = jnp.stack([jnp.pad(dv_s[:, 0], (0, NP - N)),
                     jnp.pad(dv_ns[:, 0], (0, NP - N))])
    tmp_lo, tmp_hi = _l2(src4d, dst4d, gsl, gsh, gnl, gnh, zeros)
    tmp_lo = tmp_lo.reshape(NC, NP, FW)
    tmp_hi = tmp_hi.reshape(NC, NP, FW)
    cc = _l3(src3, dst3, dv2, zerosf).reshape(NC, NP)
    b1r = b1.reshape(1, H)
    s_s = _t2(tmp_lo[0, :N], tmp_hi[0, :N], gsl, gsh, dv_s,
              dv_s.reshape(1, N), cc[0, :N].reshape(1, N), b1r)
    s_ns = _t2(tmp_lo[1, :N], tmp_hi[1, :N], gnl, gnh, dv_ns,
               dv_ns.reshape(1, N), cc[1, :N].reshape(1, N), b1r)

    onehot = jax.nn.one_hot(action, A, dtype=jnp.float32)
    f_ns, nsh, ah = _t3(s_s, s_ns, W2, b2, onehot,
                        Wf, bf, Wi1, bi1, Wi2, bi2, Wi3, bi3)
    return (f_ns, nsh, ah)
